# baseline (speedup 1.0000x reference)
import functools

import jax
import jax.numpy as jnp
from jax import lax
from jax.experimental import pallas as pl
from jax.experimental.pallas import tpu as pltpu
from jax.experimental.pallas import tpu_sc as plsc

N = 10000
E = 320000
D = 128
H = 64
C = 40
CP = 48
NC = 2
NS = 16
NT = NC * NS
B = 128
CPT = 80
NCHUNK = NT * CPT
E_PAD = NCHUNK * B
KC = 8
SPT = CPT // KC
NSC = NCHUNK // KC
ROWS_PER_SUB = 632
N_PAD = NS * ROWS_PER_SUB
DUMP_ROW = N_PAD - 1

_mesh = plsc.VectorSubcoreMesh(core_axis_name="c", subcore_axis_name="s")
_sc_params = pltpu.CompilerParams(use_tc_tiling_on_sc=False)


def _make_spmm(Wd):
  @functools.partial(
      pl.kernel,
      out_type=jax.ShapeDtypeStruct((NC, N_PAD, Wd), jnp.float32),
      mesh=_mesh,
      scratch_types=[
          pltpu.VMEM((B,), jnp.int32),
          pltpu.VMEM((B,), jnp.int32),
          pltpu.VMEM((B, Wd), jnp.float32),
          pltpu.VMEM_SHARED((N_PAD, Wd), jnp.float32),
          pltpu.SemaphoreType.DMA,
      ],
      compiler_params=_sc_params,
  )
  def spmm(w_hbm, src_hbm, dst_hbm, z_hbm, out_hbm, sb, db, rb, acc, gsem):
    c = lax.axis_index("c")
    s = lax.axis_index("s")
    wid = s * NC + c
    r0 = s * ROWS_PER_SUB

    pltpu.sync_copy(z_hbm.at[pl.ds(r0, ROWS_PER_SUB)],
                    acc.at[pl.ds(r0, ROWS_PER_SUB)])
    plsc.subcore_barrier()

    nj = jnp.where(wid < NCHUNK - (NCHUNK // NT) * NT,
                   NCHUNK // NT + 1, NCHUNK // NT)

    @pl.loop(0, nj)
    def _(j):
      base = (wid + j * NT) * B
      pltpu.sync_copy(src_hbm.at[pl.ds(base, B)], sb)
      pltpu.sync_copy(dst_hbm.at[pl.ds(base, B)], db)
      pltpu.async_copy(w_hbm.at[sb], rb, gsem).wait()
      pltpu.sync_copy(rb, acc.at[db], add=True)

    plsc.subcore_barrier()
    pltpu.sync_copy(acc.at[pl.ds(r0, ROWS_PER_SUB)],
                    out_hbm.at[c, pl.ds(r0, ROWS_PER_SUB)])

  return spmm


_spmm_h = _make_spmm(H)
_spmm_c = _make_spmm(CP)


@functools.partial(
    pl.kernel,
    out_type=jax.ShapeDtypeStruct((NC, N_PAD, 16), jnp.float32),
    mesh=_mesh,
    scratch_types=[
        pltpu.VMEM((B,), jnp.int32),
        pltpu.VMEM((B, 16), jnp.float32),
        pltpu.VMEM_SHARED((N_PAD, 16), jnp.float32),
        pltpu.SemaphoreType.DMA,
    ],
    compiler_params=_sc_params,
)
def _sc_degree(dst_hbm, z_hbm, ones_hbm, out_hbm, dst_v, ones_v, acc, sem):
  c = lax.axis_index("c")
  s = lax.axis_index("s")
  wid = s * NC + c
  r0 = s * ROWS_PER_SUB
  pltpu.sync_copy(z_hbm.at[pl.ds(r0, ROWS_PER_SUB)],
                  acc.at[pl.ds(r0, ROWS_PER_SUB)])
  pltpu.sync_copy(ones_hbm, ones_v)
  plsc.subcore_barrier()

  nj = jnp.where(wid < 2500 - (2500 // NT) * NT, 2500 // NT + 1, 2500 // NT)

  @pl.loop(0, nj)
  def _(j):
    base = (wid + j * NT) * B
    pltpu.sync_copy(dst_hbm.at[pl.ds(base, B)], dst_v)
    pltpu.async_copy(ones_v, acc.at[dst_v], sem, add=True).wait()

  plsc.subcore_barrier()
  pltpu.sync_copy(acc.at[pl.ds(r0, ROWS_PER_SUB)],
                  out_hbm.at[c, pl.ds(r0, ROWS_PER_SUB)])


def _mlp1_body(f_ref, w1_ref, b1_ref, x_ref):
  x_ref[...] = jnp.dot(f_ref[...], w1_ref[...],
                       preferred_element_type=jnp.float32) + b1_ref[...]


def _prep_body(parts_ref, x_ref, q_ref, n_ref, c1_ref):
  p = parts_ref[...]
  deg = p[0, :N, 0] + p[1, :N, 0] + 1.0
  q_ref[...] = (1.0 / deg)[:, None]
  nrm = lax.rsqrt(deg)
  n_ref[...] = nrm[:, None]
  c1_ref[...] = nrm[:, None] * x_ref[...]


def _update_body(parts_ref, w_ref, q_ref, c_ref, o_ref):
  p = parts_ref[...]
  w = w_ref[...]
  o_ref[...] = q_ref[...] * (p[0, :N] + p[1, :N] + w) - w + c_ref[...]


def _mlp2_body(w_ref, w2_ref, b2_ref, n_ref, o_ref):
  x = jnp.dot(w_ref[...], w2_ref[...], preferred_element_type=jnp.float32)
  o_ref[...] = x + n_ref[...] * b2_ref[...]


def _final_body(parts_ref, w_ref, q_ref, c_ref, n_ref, o_ref):
  p = parts_ref[...]
  w = w_ref[...]
  o_ref[...] = (q_ref[...] * (p[0, :N] + p[1, :N] + w) - w
                + c_ref[...]) / n_ref[...]


def kernel(features, edge_index, W1, b1, W2, b2):
  ei = edge_index.astype(jnp.int32)
  npad = E_PAD - E
  pad_dst = N + (jnp.arange(npad, dtype=jnp.int32) % (N_PAD - N))
  pad_src = jnp.arange(npad, dtype=jnp.int32) % N
  src1 = jnp.concatenate([ei[0], pad_src])
  dst1 = jnp.concatenate([ei[1], pad_dst])

  zH = jnp.zeros((N_PAD, H), jnp.float32)
  zC = jnp.zeros((N_PAD, CP), jnp.float32)
  z16 = jnp.zeros((N_PAD, 16), jnp.float32)
  ones = jnp.ones((B, 16), jnp.float32)
  W2p = jnp.pad(W2, ((0, 0), (0, CP - C)))
  b2p = jnp.pad(b2, ((0, CP - C),))

  deg_parts = _sc_degree(dst1, z16, ones)

  x1 = pl.pallas_call(
      _mlp1_body, out_shape=jax.ShapeDtypeStruct((N, H), jnp.float32))(
          features, W1, b1)

  q, nrm, c1 = pl.pallas_call(
      _prep_body,
      out_shape=[
          jax.ShapeDtypeStruct((N, 1), jnp.float32),
          jax.ShapeDtypeStruct((N, 1), jnp.float32),
          jax.ShapeDtypeStruct((N, H), jnp.float32),
      ],
  )(deg_parts, x1)

  update_h = pl.pallas_call(
      _update_body, out_shape=jax.ShapeDtypeStruct((N, H), jnp.float32))
  update_c = pl.pallas_call(
      _update_body, out_shape=jax.ShapeDtypeStruct((N, CP), jnp.float32))

  w = c1
  for _ in range(4):
    parts = _spmm_h(w, src1, dst1, zH)
    w = update_h(parts, w, q, c1)

  c2 = pl.pallas_call(
      _mlp2_body, out_shape=jax.ShapeDtypeStruct((N, CP), jnp.float32))(
          w, W2p, b2p, nrm)

  v = c2
  for _ in range(3):
    parts = _spmm_c(v, src1, dst1, zC)
    v = update_c(parts, v, q, c2)

  parts = _spmm_c(v, src1, dst1, zC)
  out = pl.pallas_call(
      _final_body, out_shape=jax.ShapeDtypeStruct((N, CP), jnp.float32))(
          parts, v, q, c2, nrm)

  return out[:, :C]


# speedup vs baseline: 1.9298x; 1.9298x over previous
import functools

import jax
import jax.numpy as jnp
from jax import lax
from jax.experimental import pallas as pl
from jax.experimental.pallas import tpu as pltpu
from jax.experimental.pallas import tpu_sc as plsc

N = 10000
E = 320000
D = 128
H = 64
C = 40
CP = 48
NC = 2
NS = 16
NT = NC * NS
B = 128
CPT = 80
NCHUNK = NT * CPT
E_PAD = NCHUNK * B
KC = 8
SPT = CPT // KC
NSC = NCHUNK // KC
ROWS_PER_SUB = 632
N_PAD = NS * ROWS_PER_SUB
DUMP_ROW = N_PAD - 1

_mesh = plsc.VectorSubcoreMesh(core_axis_name="c", subcore_axis_name="s")
_sc_params = pltpu.CompilerParams(use_tc_tiling_on_sc=False)


def _make_spmm(Wd):
  @functools.partial(
      pl.kernel,
      out_type=jax.ShapeDtypeStruct((NC, N_PAD, Wd), jnp.float32),
      mesh=_mesh,
      scratch_types=(
          [pltpu.VMEM((B,), jnp.int32) for _ in range(KC)]
          + [pltpu.VMEM((B,), jnp.int32) for _ in range(KC)]
          + [pltpu.VMEM((B, Wd), jnp.float32) for _ in range(KC)]
          + [pltpu.VMEM_SHARED((N_PAD, Wd), jnp.float32),
             pltpu.SemaphoreType.DMA,
             pltpu.SemaphoreType.DMA,
             pltpu.SemaphoreType.DMA]
      ),
      compiler_params=_sc_params,
  )
  def spmm(w_hbm, src_hbm, dst_hbm, z_hbm, out_hbm, *scr):
    sb = scr[:KC]
    db = scr[KC:2 * KC]
    rb = scr[2 * KC:3 * KC]
    acc, isem, gsem, ssem = scr[3 * KC:]
    c = lax.axis_index("c")
    s = lax.axis_index("s")
    wid = s * NC + c
    r0 = s * ROWS_PER_SUB

    pltpu.sync_copy(z_hbm.at[pl.ds(r0, ROWS_PER_SUB)],
                    acc.at[pl.ds(r0, ROWS_PER_SUB)])
    plsc.subcore_barrier()

    @pl.loop(0, SPT)
    def _(j):
      si, di = [], []
      for k in range(KC):
        base = (wid + (j * KC + k) * NT) * B
        si.append(pltpu.async_copy(src_hbm.at[pl.ds(base, B)], sb[k], isem))
        di.append(pltpu.async_copy(dst_hbm.at[pl.ds(base, B)], db[k], isem))
      for d in si + di:
        d.wait()
      g = [pltpu.async_copy(w_hbm.at[sb[k]], rb[k], gsem) for k in range(KC)]
      for d in g:
        d.wait()
      sc = [pltpu.async_copy(rb[k], acc.at[db[k]], ssem, add=True)
            for k in range(KC)]
      for d in sc:
        d.wait()

    plsc.subcore_barrier()
    pltpu.sync_copy(acc.at[pl.ds(r0, ROWS_PER_SUB)],
                    out_hbm.at[c, pl.ds(r0, ROWS_PER_SUB)])

  return spmm


_spmm_h = _make_spmm(H)
_spmm_c = _make_spmm(CP)


@functools.partial(
    pl.kernel,
    out_type=jax.ShapeDtypeStruct((NC, N_PAD, 16), jnp.float32),
    mesh=_mesh,
    scratch_types=[
        pltpu.VMEM((B,), jnp.int32),
        pltpu.VMEM((B, 16), jnp.float32),
        pltpu.VMEM_SHARED((N_PAD, 16), jnp.float32),
        pltpu.SemaphoreType.DMA,
    ],
    compiler_params=_sc_params,
)
def _sc_degree(dst_hbm, z_hbm, ones_hbm, out_hbm, dst_v, ones_v, acc, sem):
  c = lax.axis_index("c")
  s = lax.axis_index("s")
  wid = s * NC + c
  r0 = s * ROWS_PER_SUB
  pltpu.sync_copy(z_hbm.at[pl.ds(r0, ROWS_PER_SUB)],
                  acc.at[pl.ds(r0, ROWS_PER_SUB)])
  pltpu.sync_copy(ones_hbm, ones_v)
  plsc.subcore_barrier()

  nj = jnp.where(wid < 2500 - (2500 // NT) * NT, 2500 // NT + 1, 2500 // NT)

  @pl.loop(0, nj)
  def _(j):
    base = (wid + j * NT) * B
    pltpu.sync_copy(dst_hbm.at[pl.ds(base, B)], dst_v)
    pltpu.async_copy(ones_v, acc.at[dst_v], sem, add=True).wait()

  plsc.subcore_barrier()
  pltpu.sync_copy(acc.at[pl.ds(r0, ROWS_PER_SUB)],
                  out_hbm.at[c, pl.ds(r0, ROWS_PER_SUB)])


def _mlp1_body(f_ref, w1_ref, b1_ref, x_ref):
  x_ref[...] = jnp.dot(f_ref[...], w1_ref[...],
                       preferred_element_type=jnp.float32) + b1_ref[...]


def _prep_body(parts_ref, x_ref, q_ref, n_ref, c1_ref):
  p = parts_ref[...]
  deg = p[0, :N, 0] + p[1, :N, 0] + 1.0
  q_ref[...] = (1.0 / deg)[:, None]
  nrm = lax.rsqrt(deg)
  n_ref[...] = nrm[:, None]
  c1_ref[...] = nrm[:, None] * x_ref[...]


def _update_body(parts_ref, w_ref, q_ref, c_ref, o_ref):
  p = parts_ref[...]
  w = w_ref[...]
  o_ref[...] = q_ref[...] * (p[0, :N] + p[1, :N] + w) - w + c_ref[...]


def _mlp2_body(w_ref, w2_ref, b2_ref, n_ref, o_ref):
  x = jnp.dot(w_ref[...], w2_ref[...], preferred_element_type=jnp.float32)
  o_ref[...] = x + n_ref[...] * b2_ref[...]


def _final_body(parts_ref, w_ref, q_ref, c_ref, n_ref, o_ref):
  p = parts_ref[...]
  w = w_ref[...]
  o_ref[...] = (q_ref[...] * (p[0, :N] + p[1, :N] + w) - w
                + c_ref[...]) / n_ref[...]


def kernel(features, edge_index, W1, b1, W2, b2):
  ei = edge_index.astype(jnp.int32)
  npad = E_PAD - E
  pad_dst = N + (jnp.arange(npad, dtype=jnp.int32) % (N_PAD - N))
  pad_src = jnp.arange(npad, dtype=jnp.int32) % N
  src1 = jnp.concatenate([ei[0], pad_src])
  dst1 = jnp.concatenate([ei[1], pad_dst])

  zH = jnp.zeros((N_PAD, H), jnp.float32)
  zC = jnp.zeros((N_PAD, CP), jnp.float32)
  z16 = jnp.zeros((N_PAD, 16), jnp.float32)
  ones = jnp.ones((B, 16), jnp.float32)
  W2p = jnp.pad(W2, ((0, 0), (0, CP - C)))
  b2p = jnp.pad(b2, ((0, CP - C),))

  deg_parts = _sc_degree(dst1, z16, ones)

  x1 = pl.pallas_call(
      _mlp1_body, out_shape=jax.ShapeDtypeStruct((N, H), jnp.float32))(
          features, W1, b1)

  q, nrm, c1 = pl.pallas_call(
      _prep_body,
      out_shape=[
          jax.ShapeDtypeStruct((N, 1), jnp.float32),
          jax.ShapeDtypeStruct((N, 1), jnp.float32),
          jax.ShapeDtypeStruct((N, H), jnp.float32),
      ],
  )(deg_parts, x1)

  update_h = pl.pallas_call(
      _update_body, out_shape=jax.ShapeDtypeStruct((N, H), jnp.float32))
  update_c = pl.pallas_call(
      _update_body, out_shape=jax.ShapeDtypeStruct((N, CP), jnp.float32))

  w = c1
  for _ in range(4):
    parts = _spmm_h(w, src1, dst1, zH)
    w = update_h(parts, w, q, c1)

  c2 = pl.pallas_call(
      _mlp2_body, out_shape=jax.ShapeDtypeStruct((N, CP), jnp.float32))(
          w, W2p, b2p, nrm)

  v = c2
  for _ in range(3):
    parts = _spmm_c(v, src1, dst1, zC)
    v = update_c(parts, v, q, c2)

  parts = _spmm_c(v, src1, dst1, zC)
  out = pl.pallas_call(
      _final_body, out_shape=jax.ShapeDtypeStruct((N, CP), jnp.float32))(
          parts, v, q, c2, nrm)

  return out[:, :C]


# speedup vs baseline: 1.9651x; 1.0183x over previous
import functools

import jax
import jax.numpy as jnp
from jax import lax
from jax.experimental import pallas as pl
from jax.experimental.pallas import tpu as pltpu
from jax.experimental.pallas import tpu_sc as plsc

N = 10000
E = 320000
D = 128
H = 64
C = 40
CP = 48
NC = 2
NS = 16
NT = NC * NS
B = 128
CPT = 80
NCHUNK = NT * CPT
E_PAD = NCHUNK * B
KC = 10
SPT = CPT // KC
NSC = NCHUNK // KC
ROWS_PER_SUB = 632
N_PAD = NS * ROWS_PER_SUB
DUMP_ROW = N_PAD - 1

_mesh = plsc.VectorSubcoreMesh(core_axis_name="c", subcore_axis_name="s")
_sc_params = pltpu.CompilerParams(use_tc_tiling_on_sc=False)


def _make_spmm(Wd):
  @functools.partial(
      pl.kernel,
      out_type=jax.ShapeDtypeStruct((NC, N_PAD, Wd), jnp.float32),
      mesh=_mesh,
      scratch_types=(
          [pltpu.VMEM((B,), jnp.int32) for _ in range(KC)]
          + [pltpu.VMEM((B,), jnp.int32) for _ in range(KC)]
          + [pltpu.VMEM((B, Wd), jnp.float32) for _ in range(KC)]
          + [pltpu.VMEM_SHARED((N_PAD, Wd), jnp.float32),
             pltpu.SemaphoreType.DMA,
             pltpu.SemaphoreType.DMA,
             pltpu.SemaphoreType.DMA]
      ),
      compiler_params=_sc_params,
  )
  def spmm(w_hbm, src_hbm, dst_hbm, z_hbm, out_hbm, *scr):
    sb = scr[:KC]
    db = scr[KC:2 * KC]
    rb = scr[2 * KC:3 * KC]
    acc, isem, gsem, ssem = scr[3 * KC:]
    c = lax.axis_index("c")
    s = lax.axis_index("s")
    wid = s * NC + c
    r0 = s * ROWS_PER_SUB

    pltpu.sync_copy(z_hbm.at[pl.ds(r0, ROWS_PER_SUB)],
                    acc.at[pl.ds(r0, ROWS_PER_SUB)])
    plsc.subcore_barrier()

    @pl.loop(0, SPT)
    def _(j):
      si, di = [], []
      for k in range(KC):
        base = (wid + (j * KC + k) * NT) * B
        si.append(pltpu.async_copy(src_hbm.at[pl.ds(base, B)], sb[k], isem))
        di.append(pltpu.async_copy(dst_hbm.at[pl.ds(base, B)], db[k], isem))
      for d in si + di:
        d.wait()
      g = [pltpu.async_copy(w_hbm.at[sb[k]], rb[k], gsem) for k in range(KC)]
      for d in g:
        d.wait()
      sc = [pltpu.async_copy(rb[k], acc.at[db[k]], ssem, add=True)
            for k in range(KC)]
      for d in sc:
        d.wait()

    plsc.subcore_barrier()
    pltpu.sync_copy(acc.at[pl.ds(r0, ROWS_PER_SUB)],
                    out_hbm.at[c, pl.ds(r0, ROWS_PER_SUB)])

  return spmm


_spmm_h = _make_spmm(H)
_spmm_c = _make_spmm(CP)


@functools.partial(
    pl.kernel,
    out_type=jax.ShapeDtypeStruct((NC, N_PAD, 16), jnp.float32),
    mesh=_mesh,
    scratch_types=[
        pltpu.VMEM((B,), jnp.int32),
        pltpu.VMEM((B, 16), jnp.float32),
        pltpu.VMEM_SHARED((N_PAD, 16), jnp.float32),
        pltpu.SemaphoreType.DMA,
    ],
    compiler_params=_sc_params,
)
def _sc_degree(dst_hbm, z_hbm, ones_hbm, out_hbm, dst_v, ones_v, acc, sem):
  c = lax.axis_index("c")
  s = lax.axis_index("s")
  wid = s * NC + c
  r0 = s * ROWS_PER_SUB
  pltpu.sync_copy(z_hbm.at[pl.ds(r0, ROWS_PER_SUB)],
                  acc.at[pl.ds(r0, ROWS_PER_SUB)])
  pltpu.sync_copy(ones_hbm, ones_v)
  plsc.subcore_barrier()

  nj = jnp.where(wid < 2500 - (2500 // NT) * NT, 2500 // NT + 1, 2500 // NT)

  @pl.loop(0, nj)
  def _(j):
    base = (wid + j * NT) * B
    pltpu.sync_copy(dst_hbm.at[pl.ds(base, B)], dst_v)
    pltpu.async_copy(ones_v, acc.at[dst_v], sem, add=True).wait()

  plsc.subcore_barrier()
  pltpu.sync_copy(acc.at[pl.ds(r0, ROWS_PER_SUB)],
                  out_hbm.at[c, pl.ds(r0, ROWS_PER_SUB)])


def _mlp1_body(f_ref, w1_ref, b1_ref, x_ref):
  x_ref[...] = jnp.dot(f_ref[...], w1_ref[...],
                       preferred_element_type=jnp.float32) + b1_ref[...]


def _prep_body(parts_ref, x_ref, q_ref, n_ref, c1_ref):
  p = parts_ref[...]
  deg = p[0, :N, 0] + p[1, :N, 0] + 1.0
  q_ref[...] = (1.0 / deg)[:, None]
  nrm = lax.rsqrt(deg)
  n_ref[...] = nrm[:, None]
  c1_ref[...] = nrm[:, None] * x_ref[...]


def _update_body(parts_ref, w_ref, q_ref, c_ref, o_ref):
  p = parts_ref[...]
  w = w_ref[...]
  o_ref[...] = q_ref[...] * (p[0, :N] + p[1, :N] + w) - w + c_ref[...]


def _mlp2_body(w_ref, w2_ref, b2_ref, n_ref, o_ref):
  x = jnp.dot(w_ref[...], w2_ref[...], preferred_element_type=jnp.float32)
  o_ref[...] = x + n_ref[...] * b2_ref[...]


def _final_body(parts_ref, w_ref, q_ref, c_ref, n_ref, o_ref):
  p = parts_ref[...]
  w = w_ref[...]
  o_ref[...] = (q_ref[...] * (p[0, :N] + p[1, :N] + w) - w
                + c_ref[...]) / n_ref[...]


def kernel(features, edge_index, W1, b1, W2, b2):
  ei = edge_index.astype(jnp.int32)
  npad = E_PAD - E
  pad_dst = N + (jnp.arange(npad, dtype=jnp.int32) % (N_PAD - N))
  pad_src = jnp.arange(npad, dtype=jnp.int32) % N
  src1 = jnp.concatenate([ei[0], pad_src])
  dst1 = jnp.concatenate([ei[1], pad_dst])

  zH = jnp.zeros((N_PAD, H), jnp.float32)
  zC = jnp.zeros((N_PAD, CP), jnp.float32)
  z16 = jnp.zeros((N_PAD, 16), jnp.float32)
  ones = jnp.ones((B, 16), jnp.float32)
  W2p = jnp.pad(W2, ((0, 0), (0, CP - C)))
  b2p = jnp.pad(b2, ((0, CP - C),))

  deg_parts = _sc_degree(dst1, z16, ones)

  x1 = pl.pallas_call(
      _mlp1_body, out_shape=jax.ShapeDtypeStruct((N, H), jnp.float32))(
          features, W1, b1)

  q, nrm, c1 = pl.pallas_call(
      _prep_body,
      out_shape=[
          jax.ShapeDtypeStruct((N, 1), jnp.float32),
          jax.ShapeDtypeStruct((N, 1), jnp.float32),
          jax.ShapeDtypeStruct((N, H), jnp.float32),
      ],
  )(deg_parts, x1)

  update_h = pl.pallas_call(
      _update_body, out_shape=jax.ShapeDtypeStruct((N, H), jnp.float32))
  update_c = pl.pallas_call(
      _update_body, out_shape=jax.ShapeDtypeStruct((N, CP), jnp.float32))

  w = c1
  for _ in range(4):
    parts = _spmm_h(w, src1, dst1, zH)
    w = update_h(parts, w, q, c1)

  c2 = pl.pallas_call(
      _mlp2_body, out_shape=jax.ShapeDtypeStruct((N, CP), jnp.float32))(
          w, W2p, b2p, nrm)

  v = c2
  for _ in range(3):
    parts = _spmm_c(v, src1, dst1, zC)
    v = update_c(parts, v, q, c2)

  parts = _spmm_c(v, src1, dst1, zC)
  out = pl.pallas_call(
      _final_body, out_shape=jax.ShapeDtypeStruct((N, CP), jnp.float32))(
          parts, v, q, c2, nrm)

  return out[:, :C]
